# baseline (speedup 1.0000x reference)
import functools

import jax
import jax.numpy as jnp
from jax import lax
from jax.experimental import pallas as pl
from jax.experimental.pallas import tpu as pltpu
from jax.experimental.pallas import tpu_sc as plsc

TOKENS = 2048
DM = 768
DFF = 768
NE = 64
TILE = 64

NC = 2
NS = 16
NW = NC * NS
BPW = TOKENS // NW


def _sc_mesh():
    return plsc.VectorSubcoreMesh(
        core_axis_name="c", subcore_axis_name="s", num_cores=NC,
        num_subcores=NS)


def _gather_body(src_hbm, rw_hbm, idx_hbm, out_hbm, rw_out_hbm,
                 idx_v, rows_v, rws_v, sem, sem2):
    wid = lax.axis_index("s") * NC + lax.axis_index("c")
    base = wid * BPW
    pltpu.sync_copy(idx_hbm.at[pl.ds(base, BPW)], idx_v)
    cp = pltpu.async_copy(src_hbm.at[idx_v], rows_v, sem)
    cp2 = pltpu.async_copy(rw_hbm.at[idx_v], rws_v, sem2)
    cp2.wait()
    pltpu.sync_copy(rws_v, rw_out_hbm.at[pl.ds(base, BPW)])
    cp.wait()
    pltpu.sync_copy(rows_v, out_hbm.at[pl.ds(base, BPW)])


def _scatter_body(src_hbm, idx_hbm, out_hbm, idx_v, rows_v, sem):
    wid = lax.axis_index("s") * NC + lax.axis_index("c")
    base = wid * BPW
    pltpu.sync_copy(idx_hbm.at[pl.ds(base, BPW)], idx_v)
    pltpu.sync_copy(src_hbm.at[pl.ds(base, BPW)], rows_v)
    pltpu.async_copy(rows_v, out_hbm.at[idx_v], sem).wait()


@functools.lru_cache(maxsize=None)
def _sc_gather():
    return pl.kernel(
        _gather_body,
        out_type=[jax.ShapeDtypeStruct((TOKENS, DM), jnp.float32),
                  jax.ShapeDtypeStruct((TOKENS,), jnp.float32)],
        mesh=_sc_mesh(),
        scratch_types=[
            pltpu.VMEM((BPW,), jnp.int32),
            pltpu.VMEM((BPW, DM), jnp.float32),
            pltpu.VMEM((BPW,), jnp.float32),
            pltpu.SemaphoreType.DMA,
            pltpu.SemaphoreType.DMA,
        ],
    )


@functools.lru_cache(maxsize=None)
def _sc_scatter():
    return pl.kernel(
        _scatter_body,
        out_type=jax.ShapeDtypeStruct((TOKENS, DM), jnp.float32),
        mesh=_sc_mesh(),
        scratch_types=[
            pltpu.VMEM((BPW,), jnp.int32),
            pltpu.VMEM((BPW, DM), jnp.float32),
            pltpu.SemaphoreType.DMA,
        ],
    )


EPB = 2


def _lower_bound(ss_ref, v):
    def step(_, lohi):
        lo, hi = lohi
        mid = (lo + hi) // 2
        below = ss_ref[mid] < v
        return (jnp.where(below, mid + 1, lo), jnp.where(below, hi, mid))
    lo, _ = lax.fori_loop(0, 11, step, (jnp.int32(0), jnp.int32(TOKENS)))
    return lo


def _ffn_expert(ss_ref, x_ref, rw_ref, out_ref, gu, dp, e):
    start = _lower_bound(ss_ref, e)
    end = _lower_bound(ss_ref, e + 1)
    n = end - start
    s0 = (start // 8) * 8
    m = lax.select(n > 0, (n + (start - s0) + TILE - 1) // TILE, 0)

    def body(i, carry):
        st = jnp.minimum(s0 + i * TILE, TOKENS - TILE)
        x = x_ref[pl.ds(st, TILE), :]
        proj = lax.dot_general(x, gu, (((1,), (1,)), ((), ())),
                               preferred_element_type=jnp.float32)
        gate = proj[:, :DFF]
        up = proj[:, DFF:]
        hid = gate * jax.nn.sigmoid(gate) * up
        y = lax.dot_general(hid, dp, (((1,), (1,)), ((), ())),
                            preferred_element_type=jnp.float32)
        w = rw_ref[pl.ds(st, TILE), :][:, 0:1]
        y = y * w
        rows = st + lax.broadcasted_iota(jnp.int32, (TILE, 1), 0)
        mask = (rows >= start) & (rows < end)
        prev = out_ref[pl.ds(st, TILE), :]
        out_ref[pl.ds(st, TILE), :] = jnp.where(mask, y, prev)
        return carry

    lax.fori_loop(0, m, body, 0)


def _ffn_kernel(ss_ref, x_ref, rw_ref, gu_ref, dp_ref, out_ref):
    g = pl.program_id(0)
    for q in range(EPB):
        _ffn_expert(ss_ref, x_ref, rw_ref, out_ref,
                    gu_ref[q], dp_ref[q], g * EPB + q)


_ffn = pl.pallas_call(
    _ffn_kernel,
    grid=(NE // EPB,),
    in_specs=[
        pl.BlockSpec(memory_space=pltpu.SMEM),
        pl.BlockSpec((TOKENS, DM), lambda g: (0, 0)),
        pl.BlockSpec((TOKENS, 128), lambda g: (0, 0)),
        pl.BlockSpec((EPB, 2 * DFF, DM), lambda g: (g, 0, 0)),
        pl.BlockSpec((EPB, DM, DFF), lambda g: (g, 0, 0)),
    ],
    out_specs=pl.BlockSpec((TOKENS, DM), lambda g: (0, 0)),
    out_shape=jax.ShapeDtypeStruct((TOKENS, DM), jnp.float32),
    compiler_params=pltpu.CompilerParams(
        dimension_semantics=("arbitrary",)),
)


@jax.jit
def kernel(hidden_2d, selected_experts, routing_weights, gate_up_proj,
           down_proj):
    sel = selected_experts[:, 0].astype(jnp.int32)
    key = jnp.sort(sel * TOKENS + jnp.arange(TOKENS, dtype=jnp.int32))
    order = key & (TOKENS - 1)
    sorted_sel = key >> 11

    x_sorted, rw_sorted = _sc_gather()(hidden_2d, routing_weights[:, 0], order)
    rw_b = jnp.broadcast_to(rw_sorted[:, None], (TOKENS, 128))
    y_sorted = _ffn(sorted_sel, x_sorted, rw_b, gate_up_proj, down_proj)
    return _sc_scatter()(y_sorted, order)


# speedup vs baseline: 1.0165x; 1.0165x over previous
import functools

import jax
import jax.numpy as jnp
from jax import lax
from jax.experimental import pallas as pl
from jax.experimental.pallas import tpu as pltpu
from jax.experimental.pallas import tpu_sc as plsc

TOKENS = 2048
DM = 768
DFF = 768
NE = 64
TILE = 48

NC = 2
NS = 16
NW = NC * NS
BPW = TOKENS // NW


def _sc_mesh():
    return plsc.VectorSubcoreMesh(
        core_axis_name="c", subcore_axis_name="s", num_cores=NC,
        num_subcores=NS)


def _gather_body(src_hbm, rw_hbm, idx_hbm, out_hbm, rw_out_hbm,
                 idx_v, rows_v, rws_v, sem, sem2):
    wid = lax.axis_index("s") * NC + lax.axis_index("c")
    base = wid * BPW
    pltpu.sync_copy(idx_hbm.at[pl.ds(base, BPW)], idx_v)
    cp = pltpu.async_copy(src_hbm.at[idx_v], rows_v, sem)
    cp2 = pltpu.async_copy(rw_hbm.at[idx_v], rws_v, sem2)
    cp2.wait()
    pltpu.sync_copy(rws_v, rw_out_hbm.at[pl.ds(base, BPW)])
    cp.wait()
    pltpu.sync_copy(rows_v, out_hbm.at[pl.ds(base, BPW)])


def _scatter_body(src_hbm, idx_hbm, out_hbm, idx_v, rows_v, sem):
    wid = lax.axis_index("s") * NC + lax.axis_index("c")
    base = wid * BPW
    pltpu.sync_copy(idx_hbm.at[pl.ds(base, BPW)], idx_v)
    pltpu.sync_copy(src_hbm.at[pl.ds(base, BPW)], rows_v)
    pltpu.async_copy(rows_v, out_hbm.at[idx_v], sem).wait()


@functools.lru_cache(maxsize=None)
def _sc_gather():
    return pl.kernel(
        _gather_body,
        out_type=[jax.ShapeDtypeStruct((TOKENS, DM), jnp.float32),
                  jax.ShapeDtypeStruct((TOKENS,), jnp.float32)],
        mesh=_sc_mesh(),
        scratch_types=[
            pltpu.VMEM((BPW,), jnp.int32),
            pltpu.VMEM((BPW, DM), jnp.float32),
            pltpu.VMEM((BPW,), jnp.float32),
            pltpu.SemaphoreType.DMA,
            pltpu.SemaphoreType.DMA,
        ],
    )


@functools.lru_cache(maxsize=None)
def _sc_scatter():
    return pl.kernel(
        _scatter_body,
        out_type=jax.ShapeDtypeStruct((TOKENS, DM), jnp.float32),
        mesh=_sc_mesh(),
        scratch_types=[
            pltpu.VMEM((BPW,), jnp.int32),
            pltpu.VMEM((BPW, DM), jnp.float32),
            pltpu.SemaphoreType.DMA,
        ],
    )


EPB = 2


def _ffn_expert(off_ref, x_ref, rw_ref, out_ref, gu, dp, e):
    start = off_ref[e]
    end = off_ref[e + 1]
    n = end - start
    s0 = (start // 8) * 8
    m = lax.select(n > 0, (n + (start - s0) + TILE - 1) // TILE, 0)

    def body(i, carry):
        st = jnp.minimum(s0 + i * TILE, TOKENS - TILE)
        x = x_ref[pl.ds(st, TILE), :]
        proj = lax.dot_general(x, gu, (((1,), (1,)), ((), ())),
                               preferred_element_type=jnp.float32)
        gate = proj[:, :DFF]
        up = proj[:, DFF:]
        hid = gate * jax.nn.sigmoid(gate) * up
        y = lax.dot_general(hid, dp, (((1,), (1,)), ((), ())),
                            preferred_element_type=jnp.float32)
        w = rw_ref[pl.ds(st, TILE), :][:, 0:1]
        y = y * w
        rows = st + lax.broadcasted_iota(jnp.int32, (TILE, 1), 0)
        mask = (rows >= start) & (rows < end)
        prev = out_ref[pl.ds(st, TILE), :]
        out_ref[pl.ds(st, TILE), :] = jnp.where(mask, y, prev)
        return carry

    lax.fori_loop(0, m, body, 0)


def _ffn_kernel(off_ref, x_ref, rw_ref, gu_ref, dp_ref, out_ref):
    g = pl.program_id(0)
    for q in range(EPB):
        _ffn_expert(off_ref, x_ref, rw_ref, out_ref,
                    gu_ref[q], dp_ref[q], g * EPB + q)


_ffn = pl.pallas_call(
    _ffn_kernel,
    grid=(NE // EPB,),
    in_specs=[
        pl.BlockSpec(memory_space=pltpu.SMEM),
        pl.BlockSpec((TOKENS, DM), lambda g: (0, 0)),
        pl.BlockSpec((TOKENS, 128), lambda g: (0, 0)),
        pl.BlockSpec((EPB, 2 * DFF, DM), lambda g: (g, 0, 0)),
        pl.BlockSpec((EPB, DM, DFF), lambda g: (g, 0, 0)),
    ],
    out_specs=pl.BlockSpec((TOKENS, DM), lambda g: (0, 0)),
    out_shape=jax.ShapeDtypeStruct((TOKENS, DM), jnp.float32),
    compiler_params=pltpu.CompilerParams(
        dimension_semantics=("arbitrary",)),
)


@jax.jit
def kernel(hidden_2d, selected_experts, routing_weights, gate_up_proj,
           down_proj):
    sel = selected_experts[:, 0].astype(jnp.int32)
    key = jnp.sort(sel * TOKENS + jnp.arange(TOKENS, dtype=jnp.int32))
    order = key & (TOKENS - 1)
    sorted_sel = key >> 11
    offsets = jnp.searchsorted(
        sorted_sel, jnp.arange(NE + 1, dtype=jnp.int32),
        side="left").astype(jnp.int32)

    x_sorted, rw_sorted = _sc_gather()(hidden_2d, routing_weights[:, 0], order)
    rw_b = jnp.broadcast_to(rw_sorted[:, None], (TOKENS, 128))
    y_sorted = _ffn(offsets, x_sorted, rw_b, gate_up_proj, down_proj)
    return _sc_scatter()(y_sorted, order)
